# baseline (speedup 1.0000x reference)
import functools

import jax
import jax.numpy as jnp
from jax import lax
from jax.experimental import pallas as pl
from jax.experimental.pallas import tpu as pltpu
from jax.experimental.pallas import tpu_sc as plsc

NUM_CLASS = 1000
EMBED_DIM = 128
BATCH = 16384

_info = plsc.get_sparse_core_info()
_NC, _NS = _info.num_cores, _info.num_subcores
_NW = _NC * _NS
_B_PER_W = BATCH // _NW

_STAGE_ROWS = 64

_NBUF = 8
_CHUNK = _B_PER_W // _NBUF
_NBOUNCE = 3

_mesh = plsc.VectorSubcoreMesh(core_axis_name="c", subcore_axis_name="s")


@functools.partial(
    pl.kernel,
    mesh=_mesh,
    out_type=jax.ShapeDtypeStruct((BATCH, EMBED_DIM), jnp.float32),
    scratch_types=[
        pltpu.VMEM((_B_PER_W,), jnp.int32),
        pltpu.VMEM((_NBUF, _CHUNK, EMBED_DIM), jnp.float32),
        pltpu.VMEM_SHARED((NUM_CLASS, EMBED_DIM), jnp.float32),
        pltpu.VMEM_SHARED((_NS, _NBOUNCE * _CHUNK, EMBED_DIM), jnp.float32),
        pltpu.SemaphoreType.DMA((_NBUF,)),
        pltpu.SemaphoreType.DMA((_NBUF,)),
        pltpu.SemaphoreType.DMA,
    ],
)
def _gather_kernel(idx_hbm, table_hbm, out_hbm, idx_v, rows_v, tbl_sh, out_sh, gsem, ssem, isem):
    sid = lax.axis_index("s")
    wid = sid * _NC + lax.axis_index("c")
    base = wid * _B_PER_W
    icopy = pltpu.async_copy(idx_hbm.at[pl.ds(base, _B_PER_W)], idx_v, isem)
    n_full = NUM_CLASS // _STAGE_ROWS
    rem = NUM_CLASS - n_full * _STAGE_ROWS

    @pl.when(sid < n_full)
    def _():
        r0 = pl.multiple_of(sid * _STAGE_ROWS, _STAGE_ROWS)
        pltpu.sync_copy(
            table_hbm.at[pl.ds(r0, _STAGE_ROWS)],
            tbl_sh.at[pl.ds(r0, _STAGE_ROWS)],
        )

    @pl.when(sid == n_full)
    def _():
        r0 = n_full * _STAGE_ROWS
        pltpu.sync_copy(
            table_hbm.at[pl.ds(r0, rem)],
            tbl_sh.at[pl.ds(r0, rem)],
        )
    icopy.wait()
    gathers = [
        pltpu.async_copy(
            table_hbm.at[idx_v.at[pl.ds(0, _CHUNK)]], rows_v.at[0], gsem.at[0]
        )
    ]
    plsc.subcore_barrier()
    gathers += [
        pltpu.async_copy(
            tbl_sh.at[idx_v.at[pl.ds(b * _CHUNK, _CHUNK)]],
            rows_v.at[b],
            gsem.at[b],
        )
        for b in range(1, _NBUF)
    ]
    n_direct = _NBUF - _NBOUNCE
    scatters = []
    for b in range(_NBUF):
        gathers[b].wait()
        if b < n_direct:
            scatters.append(
                pltpu.async_copy(
                    rows_v.at[b],
                    out_hbm.at[pl.ds(base + b * _CHUNK, _CHUNK)],
                    ssem.at[b],
                )
            )
        else:
            j = b - n_direct
            pltpu.sync_copy(
                rows_v.at[b], out_sh.at[sid, pl.ds(j * _CHUNK, _CHUNK)]
            )
            scatters.append(
                pltpu.async_copy(
                    out_sh.at[sid, pl.ds(j * _CHUNK, _CHUNK)],
                    out_hbm.at[pl.ds(base + b * _CHUNK, _CHUNK)],
                    ssem.at[b],
                )
            )
    for s in scatters:
        s.wait()


def kernel(labels, embedding_table):
    labels = labels.astype(jnp.int32)
    out = _gather_kernel(labels, embedding_table)
    return out[:, None, :]


# speedup vs baseline: 1.0519x; 1.0519x over previous
import functools

import jax
import jax.numpy as jnp
from jax import lax
from jax.experimental import pallas as pl
from jax.experimental.pallas import tpu as pltpu
from jax.experimental.pallas import tpu_sc as plsc

NUM_CLASS = 1000
EMBED_DIM = 128
BATCH = 16384

_info = plsc.get_sparse_core_info()
_NC, _NS = _info.num_cores, _info.num_subcores
_NW = _NC * _NS
_B_PER_W = BATCH // _NW

_STAGE_ROWS = 64

_CHUNKS = (32, 32, 64, 64, 96, 112, 112)
_OFFS = tuple(sum(_CHUNKS[:i]) for i in range(len(_CHUNKS)))
_NBUF = len(_CHUNKS)

_mesh = plsc.VectorSubcoreMesh(core_axis_name="c", subcore_axis_name="s")


@functools.partial(
    pl.kernel,
    mesh=_mesh,
    out_type=jax.ShapeDtypeStruct((BATCH, EMBED_DIM), jnp.float32),
    scratch_types=[
        pltpu.VMEM((_B_PER_W,), jnp.int32),
        pltpu.VMEM((_B_PER_W, EMBED_DIM), jnp.float32),
        pltpu.VMEM_SHARED((NUM_CLASS, EMBED_DIM), jnp.float32),
        pltpu.SemaphoreType.DMA((_NBUF,)),
        pltpu.SemaphoreType.DMA((_NBUF,)),
        pltpu.SemaphoreType.DMA,
    ],
)
def _gather_kernel(idx_hbm, table_hbm, out_hbm, idx_v, rows_v, tbl_sh, gsem, ssem, isem):
    sid = lax.axis_index("s")
    wid = sid * _NC + lax.axis_index("c")
    base = wid * _B_PER_W
    icopy = pltpu.async_copy(idx_hbm.at[pl.ds(base, _B_PER_W)], idx_v, isem)
    n_full = NUM_CLASS // _STAGE_ROWS
    rem = NUM_CLASS - n_full * _STAGE_ROWS

    @pl.when(sid < n_full)
    def _():
        r0 = pl.multiple_of(sid * _STAGE_ROWS, _STAGE_ROWS)
        pltpu.sync_copy(
            table_hbm.at[pl.ds(r0, _STAGE_ROWS)],
            tbl_sh.at[pl.ds(r0, _STAGE_ROWS)],
        )

    @pl.when(sid == n_full)
    def _():
        r0 = n_full * _STAGE_ROWS
        pltpu.sync_copy(
            table_hbm.at[pl.ds(r0, rem)],
            tbl_sh.at[pl.ds(r0, rem)],
        )
    icopy.wait()
    gathers = [
        pltpu.async_copy(
            table_hbm.at[idx_v.at[pl.ds(0, _CHUNKS[0])]],
            rows_v.at[pl.ds(0, _CHUNKS[0])],
            gsem.at[0],
        )
    ]
    plsc.subcore_barrier()
    gathers += [
        pltpu.async_copy(
            tbl_sh.at[idx_v.at[pl.ds(_OFFS[b], _CHUNKS[b])]],
            rows_v.at[pl.ds(_OFFS[b], _CHUNKS[b])],
            gsem.at[b],
        )
        for b in range(1, _NBUF)
    ]
    scatters = []
    for b in range(_NBUF):
        gathers[b].wait()
        scatters.append(
            pltpu.async_copy(
                rows_v.at[pl.ds(_OFFS[b], _CHUNKS[b])],
                out_hbm.at[pl.ds(base + _OFFS[b], _CHUNKS[b])],
                ssem.at[b],
            )
        )
    for s in scatters:
        s.wait()


def kernel(labels, embedding_table):
    labels = labels.astype(jnp.int32)
    out = _gather_kernel(labels, embedding_table)
    return out[:, None, :]
